# baseline (speedup 1.0000x reference)
import functools

import jax
import jax.numpy as jnp
from jax.experimental import pallas as pl
from jax.experimental.pallas import tpu as pltpu

BLOCK_N = 1024


def _moe_body(xs_ref, xt_ref, wcat_ref, be1_ref, dbe_ref, wg_ref, bg_ref,
              o_ref):
    xs = xs_ref[...]
    xt = xt_ref[...]
    wg = wg_ref[...]
    bg = bg_ref[...]

    g0 = jnp.sum(xs * wg, axis=1, keepdims=True) + bg[:, 0:1]
    g1 = jnp.sum(xt * wg, axis=1, keepdims=True) + bg[:, 1:2]
    w0 = jax.nn.sigmoid(g0 - g1)

    a = jnp.concatenate([xs * w0, xt * (1.0 - w0)], axis=1)
    moe = jnp.dot(a, wcat_ref[...], preferred_element_type=jnp.float32)
    moe = moe + (w0 * dbe_ref[...] + be1_ref[...])

    m = jnp.max(moe, axis=1, keepdims=True)
    ex = jnp.exp(moe - m)
    o_ref[...] = ex / jnp.sum(ex, axis=1, keepdims=True)


@functools.partial(jax.jit, static_argnames=())
def kernel(V_S, V_T, We, be, Wg, bg):
    n, c = V_S.shape
    e = We.shape[0]
    wcat = We.reshape(e * c, c)
    be1 = be[1:2, :]
    dbe = be[0:1, :] - be[1:2, :]
    wg2d = Wg.reshape(1, c)
    bg2d = bg.reshape(1, e)
    grid = (n // BLOCK_N,)
    out = pl.pallas_call(
        _moe_body,
        grid=grid,
        in_specs=[
            pl.BlockSpec((BLOCK_N, c), lambda i: (i, 0)),
            pl.BlockSpec((BLOCK_N, c), lambda i: (i, 0)),
            pl.BlockSpec((e * c, c), lambda i: (0, 0)),
            pl.BlockSpec((1, c), lambda i: (0, 0)),
            pl.BlockSpec((1, c), lambda i: (0, 0)),
            pl.BlockSpec((1, c), lambda i: (0, 0)),
            pl.BlockSpec((1, e), lambda i: (0, 0)),
        ],
        out_specs=pl.BlockSpec((BLOCK_N, c), lambda i: (i, 0)),
        out_shape=jax.ShapeDtypeStruct((n, c), jnp.float32),
        compiler_params=pltpu.CompilerParams(
            dimension_semantics=("parallel",),
        ),
    )(V_S, V_T, wcat, be1, dbe, wg2d, bg2d)
    return out


# speedup vs baseline: 1.6666x; 1.6666x over previous
import functools

import jax
import jax.numpy as jnp
from jax.experimental import pallas as pl
from jax.experimental.pallas import tpu as pltpu

BLOCK_N = 1024


def _moe_body(xs_ref, xt_ref, wcat_ref, be1_ref, dbe_ref, wg_ref, bg_ref,
              o_ref):
    xs = xs_ref[...]
    xt = xt_ref[...]
    wg = wg_ref[...]
    bg = bg_ref[...]

    g0 = jnp.sum(xs * wg, axis=1, keepdims=True) + bg[:, 0:1]
    g1 = jnp.sum(xt * wg, axis=1, keepdims=True) + bg[:, 1:2]
    w0 = jax.nn.sigmoid(g0 - g1)

    moe = jnp.dot(xs * w0, wcat_ref[0:768, :],
                  preferred_element_type=jnp.float32)
    moe = moe + jnp.dot(xt * (1.0 - w0), wcat_ref[768:1536, :],
                        preferred_element_type=jnp.float32)
    moe = moe + (w0 * dbe_ref[...] + be1_ref[...])

    m = jnp.max(moe, axis=1, keepdims=True)
    ex = jnp.exp(moe - m)
    o_ref[...] = ex / jnp.sum(ex, axis=1, keepdims=True)


@functools.partial(jax.jit, static_argnames=())
def kernel(V_S, V_T, We, be, Wg, bg):
    n, c = V_S.shape
    e = We.shape[0]
    wcat = We.reshape(e * c, c)
    be1 = be[1:2, :]
    dbe = be[0:1, :] - be[1:2, :]
    wg2d = Wg.reshape(1, c)
    bg2d = bg.reshape(1, e)
    grid = (n // BLOCK_N,)
    out = pl.pallas_call(
        _moe_body,
        grid=grid,
        in_specs=[
            pl.BlockSpec((BLOCK_N, c), lambda i: (i, 0)),
            pl.BlockSpec((BLOCK_N, c), lambda i: (i, 0)),
            pl.BlockSpec((e * c, c), lambda i: (0, 0)),
            pl.BlockSpec((1, c), lambda i: (0, 0)),
            pl.BlockSpec((1, c), lambda i: (0, 0)),
            pl.BlockSpec((1, c), lambda i: (0, 0)),
            pl.BlockSpec((1, e), lambda i: (0, 0)),
        ],
        out_specs=pl.BlockSpec((BLOCK_N, c), lambda i: (i, 0)),
        out_shape=jax.ShapeDtypeStruct((n, c), jnp.float32),
        compiler_params=pltpu.CompilerParams(
            dimension_semantics=("parallel",),
        ),
    )(V_S, V_T, wcat, be1, dbe, wg2d, bg2d)
    return out
